# baseline (speedup 1.0000x reference)
import functools

import jax
import jax.numpy as jnp
from jax import lax
from jax.experimental import pallas as pl
from jax.experimental.pallas import tpu as pltpu
from jax.experimental.pallas import tpu_sc as plsc

N = 1048576
NC = 2
NS = 16
NW = NC * NS
C = N // NW
LANES = 16
UNROLL = 8
BLK = 128
G = C // BLK


NCHUNK = 8
GC = G // NCHUNK
EC = C // NCHUNK


def _l1_body(
    preds_hbm, targets_hbm, out_hbm,
    p_v0, p_v1, t_v0, t_v1, acc_v, psem0, psem1, tsem0, tsem1,
):
    c = lax.axis_index("c")
    s = lax.axis_index("s")
    wid = s * NC + c
    base = wid * C
    g0 = wid * G
    p_bufs = (p_v0, p_v1)
    t_bufs = (t_v0, t_v1)
    p_sems = (psem0, psem1)
    t_sems = (tsem0, tsem1)

    def start(k):
        pb = p_bufs[k % 2]
        tb = t_bufs[k % 2]
        cp = pltpu.async_copy(
            preds_hbm.at[pl.ds(base + k * EC, EC)], pb, p_sems[k % 2]
        )
        ct = pltpu.async_copy(
            targets_hbm.at[pl.ds(g0 + k * GC, GC), 0, :], tb, t_sems[k % 2]
        )
        return cp, ct

    zero = jnp.zeros((LANES,), jnp.float32)
    accs = (zero,) * UNROLL
    inflight = start(0)
    for k in range(NCHUNK):
        nxt = start(k + 1) if k + 1 < NCHUNK else None
        inflight[0].wait()
        inflight[1].wait()
        pb = p_bufs[k % 2]
        tb = t_bufs[k % 2]

        def body(i, accs, pb=pb, tb=tb):
            new = []
            for j in range(UNROLL):
                p = pb[pl.ds(i * BLK + j * LANES, LANES)]
                t0 = tb[i, pl.ds(j * LANES, LANES)]
                new.append(accs[j] + jnp.abs(p - t0))
            return tuple(new)

        accs = lax.fori_loop(0, GC, body, accs)
        inflight = nxt

    total = accs[0]
    for j in range(1, UNROLL):
        total = total + accs[j]
    acc_v[...] = total * (1.0 / N)
    pltpu.sync_copy(acc_v, out_hbm.at[wid])


_mesh = plsc.VectorSubcoreMesh(core_axis_name="c", subcore_axis_name="s")

_l1_partials = functools.partial(
    pl.kernel,
    mesh=_mesh,
    compiler_params=pltpu.CompilerParams(
        needs_layout_passes=False,
        skip_device_barrier=True,
        disable_bounds_checks=True,
        disable_semaphore_checks=True,
    ),
    out_type=jax.ShapeDtypeStruct((NW, LANES), jnp.float32),
    scratch_types=[
        pltpu.VMEM((EC,), jnp.float32),
        pltpu.VMEM((EC,), jnp.float32),
        pltpu.VMEM((GC, BLK), jnp.float32),
        pltpu.VMEM((GC, BLK), jnp.float32),
        pltpu.VMEM((LANES,), jnp.float32),
        pltpu.SemaphoreType.DMA,
        pltpu.SemaphoreType.DMA,
        pltpu.SemaphoreType.DMA,
        pltpu.SemaphoreType.DMA,
    ],
)(_l1_body)


@jax.jit
def kernel(preds1, targets1):
    t3 = jnp.transpose(jnp.reshape(targets1, (N // BLK, BLK, 2)), (0, 2, 1))
    partials = _l1_partials(preds1, t3)
    loss = jnp.sum(partials)
    return loss, jnp.reshape(loss, (1,))


# speedup vs baseline: 1.0384x; 1.0384x over previous
import functools

import jax
import jax.numpy as jnp
from jax import lax
from jax.experimental import pallas as pl
from jax.experimental.pallas import tpu as pltpu
from jax.experimental.pallas import tpu_sc as plsc

N = 1048576
NC = 2
NS = 16
NW = NC * NS
C = N // NW
LANES = 16
UNROLL = 8
BLK = 128
G = C // BLK


NCHUNK = 4
GC = G // NCHUNK
EC = C // NCHUNK


def _l1_body(
    preds_hbm, targets_hbm, out_hbm,
    p_v0, p_v1, t_v0, t_v1, acc_v, psem0, psem1, tsem0, tsem1,
):
    c = lax.axis_index("c")
    s = lax.axis_index("s")
    wid = s * NC + c
    base = wid * C
    g0 = wid * G
    p_bufs = (p_v0, p_v1)
    t_bufs = (t_v0, t_v1)
    p_sems = (psem0, psem1)
    t_sems = (tsem0, tsem1)

    def start(k):
        pb = p_bufs[k % 2]
        tb = t_bufs[k % 2]
        cp = pltpu.async_copy(
            preds_hbm.at[pl.ds(base + k * EC, EC)], pb, p_sems[k % 2]
        )
        ct = pltpu.async_copy(
            targets_hbm.at[pl.ds(g0 + k * GC, GC), 0, :], tb, t_sems[k % 2]
        )
        return cp, ct

    zero = jnp.zeros((LANES,), jnp.float32)
    accs = (zero,) * UNROLL
    inflight = start(0)
    for k in range(NCHUNK):
        nxt = start(k + 1) if k + 1 < NCHUNK else None
        inflight[0].wait()
        inflight[1].wait()
        pb = p_bufs[k % 2]
        tb = t_bufs[k % 2]

        def body(i, accs, pb=pb, tb=tb):
            new = []
            for j in range(UNROLL):
                p = pb[pl.ds(i * BLK + j * LANES, LANES)]
                t0 = tb[i, pl.ds(j * LANES, LANES)]
                new.append(accs[j] + jnp.abs(p - t0))
            return tuple(new)

        accs = lax.fori_loop(0, GC, body, accs)
        inflight = nxt

    total = accs[0]
    for j in range(1, UNROLL):
        total = total + accs[j]
    acc_v[...] = total * (1.0 / N)
    pltpu.sync_copy(acc_v, out_hbm.at[wid])


_mesh = plsc.VectorSubcoreMesh(core_axis_name="c", subcore_axis_name="s")

_l1_partials = functools.partial(
    pl.kernel,
    mesh=_mesh,
    compiler_params=pltpu.CompilerParams(
        needs_layout_passes=False,
        skip_device_barrier=True,
        disable_bounds_checks=True,
        disable_semaphore_checks=True,
    ),
    out_type=jax.ShapeDtypeStruct((NW, LANES), jnp.float32),
    scratch_types=[
        pltpu.VMEM((EC,), jnp.float32),
        pltpu.VMEM((EC,), jnp.float32),
        pltpu.VMEM((GC, BLK), jnp.float32),
        pltpu.VMEM((GC, BLK), jnp.float32),
        pltpu.VMEM((LANES,), jnp.float32),
        pltpu.SemaphoreType.DMA,
        pltpu.SemaphoreType.DMA,
        pltpu.SemaphoreType.DMA,
        pltpu.SemaphoreType.DMA,
    ],
)(_l1_body)


@jax.jit
def kernel(preds1, targets1):
    t3 = jnp.transpose(jnp.reshape(targets1, (N // BLK, BLK, 2)), (0, 2, 1))
    partials = _l1_partials(preds1, t3)
    loss = jnp.sum(partials)
    return loss, jnp.reshape(loss, (1,))
